# Initial kernel scaffold; baseline (speedup 1.0000x reference)
import jax
import jax.numpy as jnp
from jax.experimental import pallas as pl


def kernel(x_dialogues, x_acts, x_lengths, emb_weight):
    raise NotImplementedError("write your pallas kernel here")


# speedup vs baseline: 11.9530x; 11.9530x over previous
import functools

import jax
import jax.numpy as jnp
from jax import lax
from jax.experimental import pallas as pl
from jax.experimental.pallas import tpu as pltpu
from jax.experimental.pallas import tpu_sc as plsc

B, T, L, V, D = 1024, 20, 50, 100000, 128
NC, NS = 2, 16
NW = NC * NS
BPW = B // NW
CHUNK = 2 * L
NCH = (T * L) // CHUNK
UPC = CHUNK // L
NK = D // 16


def _rsqrt16(s):
    i = plsc.bitcast(s, jnp.int32)
    i = jnp.int32(0x5F3759DF) - jnp.right_shift(i, 1)
    r = plsc.bitcast(i, jnp.float32)
    for _ in range(3):
        r = r * (1.5 - 0.5 * s * r * r)
    return r


def _sc_body(dial_ref, len_ref, emb_ref, out_ref,
             idx_all, rows0_v, rows1_v, len_v, utt_v, out_v, sem0, sem1):
    c = lax.axis_index("c")
    s = lax.axis_index("s")
    wid = s * NC + c
    base = wid * BPW

    pltpu.sync_copy(dial_ref.at[pl.ds(base, BPW)], idx_all)
    pltpu.sync_copy(len_ref.at[pl.ds(base, BPW)], len_v)
    bufs = (rows0_v, rows1_v)
    sems = (sem0, sem1)

    lanes = lax.iota(jnp.int32, 16)

    def dialogue(i, score_vec, g):
        b = g * 16 + i
        rlen0 = 1.0 / len_v[b, pl.ds(0, 16)]
        rlen1 = 1.0 / len_v[b, pl.ds(T - 16, 16)]
        gathers = [None, None]
        gathers[0] = pltpu.async_copy(
            emb_ref.at[idx_all.at[b, 0]], bufs[0], sems[0])
        for cc in range(NCH):
            p = cc % 2
            if cc + 1 < NCH:
                pn = (cc + 1) % 2
                gathers[pn] = pltpu.async_copy(
                    emb_ref.at[idx_all.at[b, cc + 1]], bufs[pn], sems[pn])
            gathers[p].wait()
            rows = bufs[p]
            for u in range(UPC):
                t = UPC * cc + u
                r0 = u * L

                def jstep(j, acc, _rows=rows, _r0=r0):
                    return tuple(acc[k] + _rows[_r0 + j, pl.ds(16 * k, 16)]
                                 for k in range(NK))

                acc = tuple(rows[r0, pl.ds(16 * k, 16)] for k in range(NK))
                acc = lax.fori_loop(1, L, jstep, acc)
                inv = rlen0[t] if t < 16 else rlen1[t - (T - 16)]
                for k in range(NK):
                    utt_v[t, pl.ds(16 * k, 16)] = acc[k] * inv

        a = tuple(utt_v[0, pl.ds(16 * k, 16)] for k in range(NK))
        csum = jnp.zeros((16,), jnp.float32)
        for t in range(1, T):
            bb = tuple(utt_v[t, pl.ds(16 * k, 16)] for k in range(NK))
            vdot = a[0] * bb[0]
            vsa = a[0] * a[0]
            vsb = bb[0] * bb[0]
            for k in range(1, NK):
                vdot = vdot + a[k] * bb[k]
                vsa = vsa + a[k] * a[k]
                vsb = vsb + bb[k] * bb[k]
            dot = jnp.sum(vdot)
            sa = jnp.sum(vsa)
            sb = jnp.sum(vsb)
            sval = jnp.maximum(sa * sb, jnp.float32(1e-16))
            sv = jnp.zeros((16,), jnp.float32) + sval
            dv = jnp.zeros((16,), jnp.float32) + dot
            csum = csum + dv * _rsqrt16(sv)
            a = bb
        score = csum * jnp.float32(1.0 / (T - 1))
        return jnp.where(lanes == i, score, score_vec)

    for g in range(BPW // 16):
        sv = lax.fori_loop(0, 16, functools.partial(dialogue, g=g),
                           jnp.zeros((16,), jnp.float32))
        out_v[pl.ds(g * 16, 16)] = sv
    pltpu.sync_copy(out_v, out_ref.at[pl.ds(base, BPW)])


_sc_cosine = functools.partial(
    pl.kernel,
    out_type=jax.ShapeDtypeStruct((B,), jnp.float32),
    mesh=plsc.VectorSubcoreMesh(core_axis_name="c", subcore_axis_name="s",
                                num_cores=NC, num_subcores=NS),
    compiler_params=pltpu.CompilerParams(needs_layout_passes=False),
    scratch_types=[
        pltpu.VMEM((BPW, NCH, CHUNK), jnp.int32),
        pltpu.VMEM((CHUNK, D), jnp.float32),
        pltpu.VMEM((CHUNK, D), jnp.float32),
        pltpu.VMEM((BPW, T), jnp.float32),
        pltpu.VMEM((T, D), jnp.float32),
        pltpu.VMEM((BPW,), jnp.float32),
        pltpu.SemaphoreType.DMA,
        pltpu.SemaphoreType.DMA,
    ],
)(_sc_body)


def kernel(x_dialogues, x_acts, x_lengths, emb_weight):
    del x_acts
    dial = x_dialogues.reshape(B, NCH, CHUNK)
    return _sc_cosine(dial, x_lengths, emb_weight)


# speedup vs baseline: 11.9625x; 1.0008x over previous
import functools

import jax
import jax.numpy as jnp
from jax import lax
from jax.experimental import pallas as pl
from jax.experimental.pallas import tpu as pltpu
from jax.experimental.pallas import tpu_sc as plsc

B, T, L, V, D = 1024, 20, 50, 100000, 128
NC, NS = 2, 16
NW = NC * NS
BPW = B // NW
CHUNK = 2 * L
NCH = (T * L) // CHUNK
UPC = CHUNK // L
NK = D // 16


def _rsqrt16(s):
    i = plsc.bitcast(s, jnp.int32)
    i = jnp.int32(0x5F3759DF) - jnp.right_shift(i, 1)
    r = plsc.bitcast(i, jnp.float32)
    for _ in range(3):
        r = r * (1.5 - 0.5 * s * r * r)
    return r


def _sc_body(dial_ref, len_ref, emb_ref, out_ref,
             idx_all, rows0_v, rows1_v, len_v, out_v, sem0, sem1):
    c = lax.axis_index("c")
    s = lax.axis_index("s")
    wid = s * NC + c
    base = wid * BPW

    pltpu.sync_copy(dial_ref.at[pl.ds(base, BPW)], idx_all)
    pltpu.sync_copy(len_ref.at[pl.ds(base, BPW)], len_v)
    bufs = (rows0_v, rows1_v)
    sems = (sem0, sem1)

    lanes = lax.iota(jnp.int32, 16)

    def dialogue(i, score_vec, g):
        b = g * 16 + i
        rlen0 = 1.0 / len_v[b, pl.ds(0, 16)]
        rlen1 = 1.0 / len_v[b, pl.ds(T - 16, 16)]
        gathers = [None, None]
        gathers[0] = pltpu.async_copy(
            emb_ref.at[idx_all.at[b, 0]], bufs[0], sems[0])
        csum = jnp.zeros((16,), jnp.float32)
        prev = None
        sa = None
        for cc in range(NCH):
            p = cc % 2
            if cc + 1 < NCH:
                pn = (cc + 1) % 2
                gathers[pn] = pltpu.async_copy(
                    emb_ref.at[idx_all.at[b, cc + 1]], bufs[pn], sems[pn])
            gathers[p].wait()
            rows = bufs[p]
            for u in range(UPC):
                t = UPC * cc + u
                r0 = u * L

                def jstep(j, acc, _rows=rows, _r0=r0):
                    return tuple(acc[k] + _rows[_r0 + j, pl.ds(16 * k, 16)]
                                 for k in range(NK))

                acc = tuple(rows[r0, pl.ds(16 * k, 16)] for k in range(NK))
                acc = lax.fori_loop(1, L, jstep, acc)
                inv = rlen0[t] if t < 16 else rlen1[t - (T - 16)]
                m = tuple(acc[k] * inv for k in range(NK))
                vsb = m[0] * m[0]
                for k in range(1, NK):
                    vsb = vsb + m[k] * m[k]
                sb = jnp.sum(vsb)
                if t > 0:
                    vdot = prev[0] * m[0]
                    for k in range(1, NK):
                        vdot = vdot + prev[k] * m[k]
                    dot = jnp.sum(vdot)
                    sval = jnp.maximum(sa * sb, jnp.float32(1e-16))
                    sv = jnp.zeros((16,), jnp.float32) + sval
                    dv = jnp.zeros((16,), jnp.float32) + dot
                    csum = csum + dv * _rsqrt16(sv)
                prev = m
                sa = sb
        score = csum * jnp.float32(1.0 / (T - 1))
        return jnp.where(lanes == i, score, score_vec)

    for g in range(BPW // 16):
        sv = lax.fori_loop(0, 16, functools.partial(dialogue, g=g),
                           jnp.zeros((16,), jnp.float32))
        out_v[pl.ds(g * 16, 16)] = sv
    pltpu.sync_copy(out_v, out_ref.at[pl.ds(base, BPW)])


_sc_cosine = functools.partial(
    pl.kernel,
    out_type=jax.ShapeDtypeStruct((B,), jnp.float32),
    mesh=plsc.VectorSubcoreMesh(core_axis_name="c", subcore_axis_name="s",
                                num_cores=NC, num_subcores=NS),
    compiler_params=pltpu.CompilerParams(needs_layout_passes=False),
    scratch_types=[
        pltpu.VMEM((BPW, NCH, CHUNK), jnp.int32),
        pltpu.VMEM((CHUNK, D), jnp.float32),
        pltpu.VMEM((CHUNK, D), jnp.float32),
        pltpu.VMEM((BPW, T), jnp.float32),
        pltpu.VMEM((BPW,), jnp.float32),
        pltpu.SemaphoreType.DMA,
        pltpu.SemaphoreType.DMA,
    ],
)(_sc_body)


def kernel(x_dialogues, x_acts, x_lengths, emb_weight):
    del x_acts
    dial = x_dialogues.reshape(B, NCH, CHUNK)
    return _sc_cosine(dial, x_lengths, emb_weight)
